# baseline (speedup 1.0000x reference)
import functools

import jax
import jax.numpy as jnp
from jax import lax
from jax.experimental import pallas as pl
from jax.experimental.pallas import tpu as pltpu
from jax.experimental.pallas import tpu_sc as plsc

_NUM_ITEMS = 1000000
_D = 128
_B = 16384
_H = 64

_NC = 2
_NS = 16
_NW = _NC * _NS
_BPW = _B // _NW
_CHUNK = 128
_NCHUNK = _BPW // _CHUNK


def _sc_gather(table, idx_grouped):
    mesh = plsc.VectorSubcoreMesh(core_axis_name="c", subcore_axis_name="s")

    @functools.partial(
        pl.kernel,
        mesh=mesh,
        out_type=jax.ShapeDtypeStruct((_B, _D), jnp.float32),
        scratch_types=[
            pltpu.VMEM((_NCHUNK, _CHUNK), jnp.int32),
            pltpu.VMEM((_BPW, _D), jnp.float32),
            pltpu.SemaphoreType.DMA,
        ],
    )
    def k(table_hbm, idx_hbm, out_hbm, idx_v, rows_v, sem):
        wid = lax.axis_index("s") * _NC + lax.axis_index("c")
        pltpu.sync_copy(idx_hbm.at[wid], idx_v)
        copies = []
        for j in range(_NCHUNK):
            copies.append(
                pltpu.async_copy(
                    table_hbm.at[idx_v.at[j]],
                    rows_v.at[pl.ds(j * _CHUNK, _CHUNK)],
                    sem,
                )
            )
        for c in copies:
            c.wait()
        pltpu.sync_copy(rows_v, out_hbm.at[pl.ds(wid * _BPW, _BPW)])

    return k(table, idx_grouped)


_BLK = 1024


def _tc_mlp_add(ts2d, W1, b1, W2, b2, gathered):
    def body(t_ref, w1_ref, b1_ref, w2_ref, b2_ref, g_ref, o_ref):
        t = t_ref[...]
        h = jnp.maximum(t * w1_ref[...] + b1_ref[...], 0.0)
        s = jnp.tanh(
            jnp.dot(h, w2_ref[...], preferred_element_type=jnp.float32)
            + b2_ref[...]
        )
        o_ref[...] = g_ref[...] + s

    grid = (_B // _BLK,)
    return pl.pallas_call(
        body,
        grid=grid,
        in_specs=[
            pl.BlockSpec((_BLK, 1), lambda i: (i, 0)),
            pl.BlockSpec((1, _H), lambda i: (0, 0)),
            pl.BlockSpec((1, _H), lambda i: (0, 0)),
            pl.BlockSpec((_H, _D), lambda i: (0, 0)),
            pl.BlockSpec((1, _D), lambda i: (0, 0)),
            pl.BlockSpec((_BLK, _D), lambda i: (i, 0)),
        ],
        out_specs=pl.BlockSpec((_BLK, _D), lambda i: (i, 0)),
        out_shape=jax.ShapeDtypeStruct((_B, _D), jnp.float32),
        compiler_params=pltpu.CompilerParams(
            dimension_semantics=("parallel",),
        ),
    )(ts2d, W1, b1, W2, b2, gathered)


def kernel(indices, timestamps, base_embeddings, W1, b1, W2, b2):
    idx_grouped = indices.astype(jnp.int32).reshape(_NW, _NCHUNK, _CHUNK)
    gathered = _sc_gather(base_embeddings, idx_grouped)
    ts2d = timestamps.astype(jnp.float32).reshape(_B, 1)
    return _tc_mlp_add(
        ts2d,
        W1,
        b1.reshape(1, _H),
        W2,
        b2.reshape(1, _D),
        gathered,
    )


# speedup vs baseline: 1.0734x; 1.0734x over previous
import functools

import jax
import jax.numpy as jnp
from jax import lax
from jax.experimental import pallas as pl
from jax.experimental.pallas import tpu as pltpu
from jax.experimental.pallas import tpu_sc as plsc

_NUM_ITEMS = 1000000
_D = 128
_B = 16384
_H = 64

_NC = 2
_NS = 16
_NW = _NC * _NS
_BPW = _B // _NW
_CHUNK = 128
_NCHUNK = _BPW // _CHUNK


def _sc_gather(table, idx_grouped):
    mesh = plsc.VectorSubcoreMesh(core_axis_name="c", subcore_axis_name="s")

    @functools.partial(
        pl.kernel,
        mesh=mesh,
        out_type=jax.ShapeDtypeStruct((_B, _D), jnp.float32),
        scratch_types=[
            pltpu.VMEM((_NCHUNK, _CHUNK), jnp.int32),
            pltpu.VMEM((_BPW, _D), jnp.float32),
            pltpu.SemaphoreType.DMA,
        ],
    )
    def k(table_hbm, idx_hbm, out_hbm, idx_v, rows_v, sem):
        wid = lax.axis_index("s") * _NC + lax.axis_index("c")
        pltpu.sync_copy(idx_hbm.at[wid], idx_v)
        copies = []
        for j in range(_NCHUNK):
            copies.append(
                pltpu.async_copy(
                    table_hbm.at[idx_v.at[j]],
                    rows_v.at[pl.ds(j * _CHUNK, _CHUNK)],
                    sem,
                )
            )
        for c in copies:
            c.wait()
        pltpu.sync_copy(rows_v, out_hbm.at[pl.ds(wid * _BPW, _BPW)])

    return k(table, idx_grouped)


_BLK = 1024
_GRID = _B // _BLK


def _tc_mlp_add(ts3d, W1t, b1t, W2, b2, gathered):
    def body(t_ref, w1_ref, b1_ref, w2_ref, b2_ref, g_ref, o_ref):
        trow = t_ref[0]
        ht = jnp.maximum(w1_ref[...] * trow + b1_ref[...], 0.0)
        s = jnp.tanh(
            lax.dot_general(
                ht, w2_ref[...], (((0,), (0,)), ((), ())),
                preferred_element_type=jnp.float32,
            )
            + b2_ref[...]
        )
        o_ref[...] = g_ref[...] + s

    return pl.pallas_call(
        body,
        grid=(_GRID,),
        in_specs=[
            pl.BlockSpec((1, 1, _BLK), lambda i: (i, 0, 0)),
            pl.BlockSpec((_H, 1), lambda i: (0, 0)),
            pl.BlockSpec((_H, 1), lambda i: (0, 0)),
            pl.BlockSpec((_H, _D), lambda i: (0, 0)),
            pl.BlockSpec((1, _D), lambda i: (0, 0)),
            pl.BlockSpec((_BLK, _D), lambda i: (i, 0)),
        ],
        out_specs=pl.BlockSpec((_BLK, _D), lambda i: (i, 0)),
        out_shape=jax.ShapeDtypeStruct((_B, _D), jnp.float32),
        compiler_params=pltpu.CompilerParams(
            dimension_semantics=("parallel",),
        ),
    )(ts3d, W1t, b1t, W2, b2, gathered)


def kernel(indices, timestamps, base_embeddings, W1, b1, W2, b2):
    idx_grouped = indices.astype(jnp.int32).reshape(_NW, _NCHUNK, _CHUNK)
    gathered = _sc_gather(base_embeddings, idx_grouped)
    ts3d = timestamps.astype(jnp.float32).reshape(_GRID, 1, _BLK)
    return _tc_mlp_add(
        ts3d,
        W1.reshape(_H, 1),
        b1.reshape(_H, 1),
        W2,
        b2.reshape(1, _D),
        gathered,
    )


# speedup vs baseline: 1.1889x; 1.1076x over previous
import functools

import jax
import jax.numpy as jnp
from jax import lax
from jax.experimental import pallas as pl
from jax.experimental.pallas import tpu as pltpu
from jax.experimental.pallas import tpu_sc as plsc

_NUM_ITEMS = 1000000
_D = 128
_B = 16384
_H = 64

_NC = 2
_NS = 16
_NW = _NC * _NS
_BPW = _B // _NW
_CHUNK = 128
_NCHUNK = _BPW // _CHUNK


def _sc_gather(table, idx_grouped):
    mesh = plsc.VectorSubcoreMesh(core_axis_name="c", subcore_axis_name="s")

    @functools.partial(
        pl.kernel,
        mesh=mesh,
        out_type=jax.ShapeDtypeStruct((_B, _D), jnp.float32),
        scratch_types=[
            pltpu.VMEM((_NCHUNK, _CHUNK), jnp.int32),
            pltpu.VMEM((_BPW, _D), jnp.float32),
            pltpu.SemaphoreType.DMA,
        ],
    )
    def k(table_hbm, idx_hbm, out_hbm, idx_v, rows_v, sem):
        wid = lax.axis_index("s") * _NC + lax.axis_index("c")
        pltpu.sync_copy(idx_hbm.at[wid], idx_v)
        copies = []
        for j in range(_NCHUNK):
            copies.append(
                pltpu.async_copy(
                    table_hbm.at[idx_v.at[j]],
                    rows_v.at[pl.ds(j * _CHUNK, _CHUNK)],
                    sem,
                )
            )
        for c in copies:
            c.wait()
        pltpu.sync_copy(rows_v, out_hbm.at[pl.ds(wid * _BPW, _BPW)])

    return k(table, idx_grouped)


_BLK = 2048
_GRID = _B // _BLK


def _tc_mlp_add(ts3d, W1t, b1t, W2, b2, gathered):
    def body(t_ref, w1_ref, b1_ref, w2_ref, b2_ref, g_ref, o_ref):
        trow = t_ref[0]
        ht = jnp.maximum(w1_ref[...] * trow + b1_ref[...], 0.0)
        s = jnp.tanh(
            lax.dot_general(
                ht, w2_ref[...], (((0,), (0,)), ((), ())),
                preferred_element_type=jnp.float32,
            )
            + b2_ref[...]
        )
        o_ref[...] = g_ref[...] + s

    return pl.pallas_call(
        body,
        grid=(_GRID,),
        in_specs=[
            pl.BlockSpec((1, 1, _BLK), lambda i: (i, 0, 0)),
            pl.BlockSpec((_H, 1), lambda i: (0, 0)),
            pl.BlockSpec((_H, 1), lambda i: (0, 0)),
            pl.BlockSpec((_H, _D), lambda i: (0, 0)),
            pl.BlockSpec((1, _D), lambda i: (0, 0)),
            pl.BlockSpec((_BLK, _D), lambda i: (i, 0)),
        ],
        out_specs=pl.BlockSpec((_BLK, _D), lambda i: (i, 0)),
        out_shape=jax.ShapeDtypeStruct((_B, _D), jnp.float32),
        compiler_params=pltpu.CompilerParams(
            dimension_semantics=("parallel",),
        ),
    )(ts3d, W1t, b1t, W2, b2, gathered)


def kernel(indices, timestamps, base_embeddings, W1, b1, W2, b2):
    idx_grouped = indices.astype(jnp.int32).reshape(_NW, _NCHUNK, _CHUNK)
    gathered = _sc_gather(base_embeddings, idx_grouped)
    ts3d = timestamps.astype(jnp.float32).reshape(_GRID, 1, _BLK)
    return _tc_mlp_add(
        ts3d,
        W1.reshape(_H, 1),
        b1.reshape(_H, 1),
        W2,
        b2.reshape(1, _D),
        gathered,
    )


# speedup vs baseline: 1.2818x; 1.0782x over previous
import functools

import jax
import jax.numpy as jnp
from jax import lax
from jax.experimental import pallas as pl
from jax.experimental.pallas import tpu as pltpu
from jax.experimental.pallas import tpu_sc as plsc

_NUM_ITEMS = 1000000
_D = 128
_B = 16384
_H = 64

_NC = 2
_NS = 16
_NW = _NC * _NS
_BPW = _B // _NW
_CHUNK = 128
_NCHUNK = _BPW // _CHUNK


def _sc_gather(table, idx_grouped):
    mesh = plsc.VectorSubcoreMesh(core_axis_name="c", subcore_axis_name="s")

    @functools.partial(
        pl.kernel,
        mesh=mesh,
        out_type=jax.ShapeDtypeStruct((_B, _D), jnp.float32),
        scratch_types=[
            pltpu.VMEM((_NCHUNK, _CHUNK), jnp.int32),
            pltpu.VMEM((_BPW, _D), jnp.float32),
            pltpu.SemaphoreType.DMA,
        ],
    )
    def k(table_hbm, idx_hbm, out_hbm, idx_v, rows_v, sem):
        wid = lax.axis_index("s") * _NC + lax.axis_index("c")
        pltpu.sync_copy(idx_hbm.at[wid], idx_v)
        copies = []
        for j in range(_NCHUNK):
            copies.append(
                pltpu.async_copy(
                    table_hbm.at[idx_v.at[j]],
                    rows_v.at[pl.ds(j * _CHUNK, _CHUNK)],
                    sem,
                )
            )
        for c in copies:
            c.wait()
        pltpu.sync_copy(rows_v, out_hbm.at[pl.ds(wid * _BPW, _BPW)])

    return k(table, idx_grouped)


_BLK = 4096
_GRID = _B // _BLK


def _tc_mlp_add(ts3d, W1t, b1t, W2, b2, gathered):
    def body(t_ref, w1_ref, b1_ref, w2_ref, b2_ref, g_ref, o_ref):
        trow = t_ref[0]
        ht = jnp.maximum(w1_ref[...] * trow + b1_ref[...], 0.0)
        s = jnp.tanh(
            lax.dot_general(
                ht, w2_ref[...], (((0,), (0,)), ((), ())),
                preferred_element_type=jnp.float32,
            )
            + b2_ref[...]
        )
        o_ref[...] = g_ref[...] + s

    return pl.pallas_call(
        body,
        grid=(_GRID,),
        in_specs=[
            pl.BlockSpec((1, 1, _BLK), lambda i: (i, 0, 0)),
            pl.BlockSpec((_H, 1), lambda i: (0, 0)),
            pl.BlockSpec((_H, 1), lambda i: (0, 0)),
            pl.BlockSpec((_H, _D), lambda i: (0, 0)),
            pl.BlockSpec((1, _D), lambda i: (0, 0)),
            pl.BlockSpec((_BLK, _D), lambda i: (i, 0)),
        ],
        out_specs=pl.BlockSpec((_BLK, _D), lambda i: (i, 0)),
        out_shape=jax.ShapeDtypeStruct((_B, _D), jnp.float32),
        compiler_params=pltpu.CompilerParams(
            dimension_semantics=("parallel",),
        ),
    )(ts3d, W1t, b1t, W2, b2, gathered)


def kernel(indices, timestamps, base_embeddings, W1, b1, W2, b2):
    idx_grouped = indices.astype(jnp.int32).reshape(_NW, _NCHUNK, _CHUNK)
    gathered = _sc_gather(base_embeddings, idx_grouped)
    ts3d = timestamps.astype(jnp.float32).reshape(_GRID, 1, _BLK)
    return _tc_mlp_add(
        ts3d,
        W1.reshape(_H, 1),
        b1.reshape(_H, 1),
        W2,
        b2.reshape(1, _D),
        gathered,
    )


# speedup vs baseline: 1.3362x; 1.0424x over previous
import functools

import jax
import jax.numpy as jnp
from jax import lax
from jax.experimental import pallas as pl
from jax.experimental.pallas import tpu as pltpu
from jax.experimental.pallas import tpu_sc as plsc

_NUM_ITEMS = 1000000
_D = 128
_B = 16384
_H = 64

_NC = 2
_NS = 16
_NW = _NC * _NS
_BPW = _B // _NW
_CHUNK = 128
_NCHUNK = _BPW // _CHUNK


def _sc_gather(table, idx_grouped):
    mesh = plsc.VectorSubcoreMesh(core_axis_name="c", subcore_axis_name="s")

    @functools.partial(
        pl.kernel,
        mesh=mesh,
        out_type=jax.ShapeDtypeStruct((_B, _D), jnp.float32),
        scratch_types=[
            pltpu.VMEM((_NCHUNK, _CHUNK), jnp.int32),
            pltpu.VMEM((_BPW, _D), jnp.float32),
            pltpu.SemaphoreType.DMA,
        ],
    )
    def k(table_hbm, idx_hbm, out_hbm, idx_v, rows_v, sem):
        wid = lax.axis_index("s") * _NC + lax.axis_index("c")
        pltpu.sync_copy(idx_hbm.at[wid], idx_v)
        copies = []
        for j in range(_NCHUNK):
            copies.append(
                pltpu.async_copy(
                    table_hbm.at[idx_v.at[j]],
                    rows_v.at[pl.ds(j * _CHUNK, _CHUNK)],
                    sem,
                )
            )
        for c in copies:
            c.wait()
        pltpu.sync_copy(rows_v, out_hbm.at[pl.ds(wid * _BPW, _BPW)])

    return k(table, idx_grouped)


_BLK = 8192
_GRID = _B // _BLK


def _tc_mlp_add(ts3d, W1t, b1t, W2, b2, gathered):
    def body(t_ref, w1_ref, b1_ref, w2_ref, b2_ref, g_ref, o_ref):
        trow = t_ref[0]
        ht = jnp.maximum(w1_ref[...] * trow + b1_ref[...], 0.0)
        s = jnp.tanh(
            lax.dot_general(
                ht, w2_ref[...], (((0,), (0,)), ((), ())),
                preferred_element_type=jnp.float32,
            )
            + b2_ref[...]
        )
        o_ref[...] = g_ref[...] + s

    return pl.pallas_call(
        body,
        grid=(_GRID,),
        in_specs=[
            pl.BlockSpec((1, 1, _BLK), lambda i: (i, 0, 0)),
            pl.BlockSpec((_H, 1), lambda i: (0, 0)),
            pl.BlockSpec((_H, 1), lambda i: (0, 0)),
            pl.BlockSpec((_H, _D), lambda i: (0, 0)),
            pl.BlockSpec((1, _D), lambda i: (0, 0)),
            pl.BlockSpec((_BLK, _D), lambda i: (i, 0)),
        ],
        out_specs=pl.BlockSpec((_BLK, _D), lambda i: (i, 0)),
        out_shape=jax.ShapeDtypeStruct((_B, _D), jnp.float32),
        compiler_params=pltpu.CompilerParams(
            dimension_semantics=("parallel",),
        ),
    )(ts3d, W1t, b1t, W2, b2, gathered)


def kernel(indices, timestamps, base_embeddings, W1, b1, W2, b2):
    idx_grouped = indices.astype(jnp.int32).reshape(_NW, _NCHUNK, _CHUNK)
    gathered = _sc_gather(base_embeddings, idx_grouped)
    ts3d = timestamps.astype(jnp.float32).reshape(_GRID, 1, _BLK)
    return _tc_mlp_add(
        ts3d,
        W1.reshape(_H, 1),
        b1.reshape(_H, 1),
        W2,
        b2.reshape(1, _D),
        gathered,
    )
